# baseline (speedup 1.0000x reference)
import functools

import jax
import jax.numpy as jnp
from jax import lax
from jax.experimental import pallas as pl
from jax.experimental.pallas import tpu as pltpu

_R = 64
_B = 128
_N = 10000
_K = 20

_CB = 5120
_NCOLS = _B * _N

_BB2 = 32


def _score_body(wb_ref, gx_ref, out_ref):
    xb = gx_ref[...].astype(jnp.bfloat16)
    wb = wb_ref[...]
    out_ref[...] = lax.dot_general(
        wb, xb, (((1,), (0,)), ((), ())),
        preferred_element_type=jnp.float32,
    )


def _score(gx2, wb):
    return pl.pallas_call(
        _score_body,
        grid=(_NCOLS // _CB,),
        in_specs=[
            pl.BlockSpec((1, _R), lambda j: (0, 0)),
            pl.BlockSpec((_R, _CB), lambda j: (0, j)),
        ],
        out_specs=pl.BlockSpec((1, _CB), lambda j: (0, j)),
        out_shape=jax.ShapeDtypeStruct((1, _NCOLS), jnp.float32),
        compiler_params=pltpu.CompilerParams(
            dimension_semantics=("arbitrary",),
        ),
    )(wb, gx2)


def _topk_body(raw_ref, bias_ref, score_ref, vals_ref, idx_ref):
    s = raw_ref[...] + bias_ref[...]
    score_ref[...] = s
    col = lax.broadcasted_iota(jnp.int32, (_BB2, _N), 1)
    kcol = lax.broadcasted_iota(jnp.int32, (_BB2, _K), 1)
    vals = jnp.zeros((_BB2, _K), jnp.float32)
    idxs = jnp.zeros((_BB2, _K), jnp.int32)
    for i in range(_K):
        m = jnp.max(s, axis=1, keepdims=True)
        cand = jnp.where(s == m, col, jnp.int32(2**30))
        ix = jnp.min(cand, axis=1, keepdims=True)
        vals = jnp.where(kcol == i, m, vals)
        idxs = jnp.where(kcol == i, ix, idxs)
        s = jnp.where(col == ix, -jnp.inf, s)
    vals_ref[...] = vals
    idx_ref[...] = idxs


def _topk(raw, bias):
    return pl.pallas_call(
        _topk_body,
        grid=(_B // _BB2,),
        in_specs=[
            pl.BlockSpec((_BB2, _N), lambda i: (i, 0)),
            pl.BlockSpec((1, _N), lambda i: (0, 0)),
        ],
        out_specs=[
            pl.BlockSpec((_BB2, _N), lambda i: (i, 0)),
            pl.BlockSpec((_BB2, _K), lambda i: (i, 0)),
            pl.BlockSpec((_BB2, _K), lambda i: (i, 0)),
        ],
        out_shape=[
            jax.ShapeDtypeStruct((_B, _N), jnp.float32),
            jax.ShapeDtypeStruct((_B, _K), jnp.float32),
            jax.ShapeDtypeStruct((_B, _K), jnp.int32),
        ],
    )(raw, bias.reshape(1, _N))


def kernel(grounding_x, rule_weights, bias, all_h, all_r, k):
    gx2 = grounding_x.reshape(_R, _NCOLS)
    wb = rule_weights.astype(jnp.bfloat16).reshape(1, _R)
    raw = _score(gx2, wb).reshape(_B, _N)
    score, top_vals, top_idx = _topk(raw, bias)
    mask = jnp.ones((_B, _N), dtype=jnp.bool_)
    return score, mask, top_vals, top_idx


# speedup vs baseline: 16.4329x; 16.4329x over previous
import functools

import jax
import jax.numpy as jnp
from jax import lax
from jax.experimental import pallas as pl
from jax.experimental.pallas import tpu as pltpu

_R = 64
_B = 128
_N = 10000
_K = 20

_BB = 16
_NB = 1024
_NBLK = (_N + _NB - 1) // _NB

_BB2 = 32


def _score_body(w_ref, gx_ref, out_ref):
    xb = gx_ref[...].astype(jnp.bfloat16)
    x2 = xb.reshape(_R, _BB * _NB)
    wb = w_ref[...]
    out_ref[...] = lax.dot_general(
        wb, x2, (((1,), (0,)), ((), ())),
        preferred_element_type=jnp.float32,
    ).reshape(_BB, _NB)


def _score(gx, wb):
    return pl.pallas_call(
        _score_body,
        grid=(_B // _BB, _NBLK),
        in_specs=[
            pl.BlockSpec((1, _R), lambda i, j: (0, 0)),
            pl.BlockSpec((_R, _BB, _NB), lambda i, j: (0, i, j)),
        ],
        out_specs=pl.BlockSpec((_BB, _NB), lambda i, j: (i, j)),
        out_shape=jax.ShapeDtypeStruct((_B, _N), jnp.float32),
        compiler_params=pltpu.CompilerParams(
            dimension_semantics=("parallel", "parallel"),
        ),
    )(wb, gx)


def _topk_body(raw_ref, bias_ref, score_ref, vals_ref, idx_ref):
    s = raw_ref[...] + bias_ref[...]
    score_ref[...] = s
    col = lax.broadcasted_iota(jnp.int32, (_BB2, _N), 1)
    kcol = lax.broadcasted_iota(jnp.int32, (_BB2, _K), 1)
    vals = jnp.zeros((_BB2, _K), jnp.float32)
    idxs = jnp.zeros((_BB2, _K), jnp.int32)
    for i in range(_K):
        m = jnp.max(s, axis=1, keepdims=True)
        cand = jnp.where(s == m, col, jnp.int32(2**30))
        ix = jnp.min(cand, axis=1, keepdims=True)
        vals = jnp.where(kcol == i, m, vals)
        idxs = jnp.where(kcol == i, ix, idxs)
        s = jnp.where(col == ix, -jnp.inf, s)
    vals_ref[...] = vals
    idx_ref[...] = idxs


def _topk(raw, bias):
    return pl.pallas_call(
        _topk_body,
        grid=(_B // _BB2,),
        in_specs=[
            pl.BlockSpec((_BB2, _N), lambda i: (i, 0)),
            pl.BlockSpec((1, _N), lambda i: (0, 0)),
        ],
        out_specs=[
            pl.BlockSpec((_BB2, _N), lambda i: (i, 0)),
            pl.BlockSpec((_BB2, _K), lambda i: (i, 0)),
            pl.BlockSpec((_BB2, _K), lambda i: (i, 0)),
        ],
        out_shape=[
            jax.ShapeDtypeStruct((_B, _N), jnp.float32),
            jax.ShapeDtypeStruct((_B, _K), jnp.float32),
            jax.ShapeDtypeStruct((_B, _K), jnp.int32),
        ],
    )(raw, bias.reshape(1, _N))


def kernel(grounding_x, rule_weights, bias, all_h, all_r, k):
    wb = rule_weights.astype(jnp.bfloat16).reshape(1, _R)
    raw = _score(grounding_x, wb)
    score, top_vals, top_idx = _topk(raw, bias)
    mask = jnp.ones((_B, _N), dtype=jnp.bool_)
    return score, mask, top_vals, top_idx


# speedup vs baseline: 37.4285x; 2.2777x over previous
import functools

import jax
import jax.numpy as jnp
from jax import lax
from jax.experimental import pallas as pl
from jax.experimental.pallas import tpu as pltpu

_R = 64
_B = 128
_N = 10000
_K = 20

_NB = 80
_NBLK = _N // _NB


def _score_body(w_ref, gx_ref, bias_ref, out_ref):
    xb = gx_ref[...].astype(jnp.bfloat16)
    x2 = xb.reshape(_R, _NB * _B)
    wb = w_ref[...]
    acc = lax.dot_general(
        wb, x2, (((1,), (0,)), ((), ())),
        preferred_element_type=jnp.float32,
    ).reshape(_NB, _B)
    out_ref[...] = acc + bias_ref[...]


def _score(gx_t, wb, bias2):
    return pl.pallas_call(
        _score_body,
        grid=(_NBLK,),
        in_specs=[
            pl.BlockSpec((1, _R), lambda j: (0, 0)),
            pl.BlockSpec((_R, _NB, _B), lambda j: (0, j, 0)),
            pl.BlockSpec((_NB, 1), lambda j: (j, 0)),
        ],
        out_specs=pl.BlockSpec((_NB, _B), lambda j: (j, 0)),
        out_shape=jax.ShapeDtypeStruct((_N, _B), jnp.float32),
        compiler_params=pltpu.CompilerParams(
            dimension_semantics=("arbitrary",),
        ),
    )(wb, gx_t, bias2)


def _topk_body(s_ref, vals_ref, idx_ref):
    s = s_ref[...]
    row = lax.broadcasted_iota(jnp.int32, (_N, _B), 0)
    krow = lax.broadcasted_iota(jnp.int32, (_K, _B), 0)
    vals = jnp.zeros((_K, _B), jnp.float32)
    idxs = jnp.zeros((_K, _B), jnp.int32)
    for i in range(_K):
        m = jnp.max(s, axis=0, keepdims=True)
        cand = jnp.where(s == m, row, jnp.int32(2**30))
        ix = jnp.min(cand, axis=0, keepdims=True)
        vals = jnp.where(krow == i, m, vals)
        idxs = jnp.where(krow == i, ix, idxs)
        s = jnp.where(row == ix, -jnp.inf, s)
    vals_ref[...] = vals
    idx_ref[...] = idxs


def _topk(score_t):
    return pl.pallas_call(
        _topk_body,
        grid=(1,),
        in_specs=[pl.BlockSpec((_N, _B), lambda i: (0, 0))],
        out_specs=[
            pl.BlockSpec((_K, _B), lambda i: (0, 0)),
            pl.BlockSpec((_K, _B), lambda i: (0, 0)),
        ],
        out_shape=[
            jax.ShapeDtypeStruct((_K, _B), jnp.float32),
            jax.ShapeDtypeStruct((_K, _B), jnp.int32),
        ],
    )(score_t)


def kernel(grounding_x, rule_weights, bias, all_h, all_r, k):
    gx_t = jnp.transpose(grounding_x, (0, 2, 1))
    wb = rule_weights.astype(jnp.bfloat16).reshape(1, _R)
    bias2 = bias.reshape(_N, 1)
    score_t = _score(gx_t, wb, bias2)
    vals_t, idx_t = _topk(score_t)
    mask = jnp.ones((_B, _N), dtype=jnp.bool_)
    return score_t.T, mask, vals_t.T, idx_t.T


# speedup vs baseline: 47.3072x; 1.2639x over previous
import functools

import jax
import jax.numpy as jnp
from jax import lax
from jax.experimental import pallas as pl
from jax.experimental.pallas import tpu as pltpu

_R = 64
_B = 128
_N = 10000
_K = 20

_NB = 200
_NBLK = _N // _NB


def _score_body(w_ref, gx_ref, bias_ref, out_ref):
    xb = gx_ref[...].astype(jnp.bfloat16)
    x2 = xb.reshape(_R, _NB * _B)
    wb = w_ref[...]
    acc = lax.dot_general(
        wb, x2, (((1,), (0,)), ((), ())),
        preferred_element_type=jnp.float32,
    ).reshape(_NB, _B)
    out_ref[...] = acc + bias_ref[...]


def _score(gx_t, wb, bias2):
    return pl.pallas_call(
        _score_body,
        grid=(_NBLK,),
        in_specs=[
            pl.BlockSpec((1, _R), lambda j: (0, 0)),
            pl.BlockSpec((_R, _NB, _B), lambda j: (0, j, 0)),
            pl.BlockSpec((_NB, 1), lambda j: (j, 0)),
        ],
        out_specs=pl.BlockSpec((_NB, _B), lambda j: (j, 0)),
        out_shape=jax.ShapeDtypeStruct((_N, _B), jnp.float32),
        compiler_params=pltpu.CompilerParams(
            dimension_semantics=("arbitrary",),
        ),
    )(wb, gx_t, bias2)


def _topk_body(s_ref, vals_ref, idx_ref):
    s = s_ref[...]
    row = lax.broadcasted_iota(jnp.int32, (_N, _B), 0)
    krow = lax.broadcasted_iota(jnp.int32, (_K, _B), 0)
    vals = jnp.zeros((_K, _B), jnp.float32)
    idxs = jnp.zeros((_K, _B), jnp.int32)
    for i in range(_K):
        m = jnp.max(s, axis=0, keepdims=True)
        cand = jnp.where(s == m, row, jnp.int32(2**30))
        ix = jnp.min(cand, axis=0, keepdims=True)
        vals = jnp.where(krow == i, m, vals)
        idxs = jnp.where(krow == i, ix, idxs)
        s = jnp.where(row == ix, -jnp.inf, s)
    vals_ref[...] = vals
    idx_ref[...] = idxs


def _topk(score_t):
    return pl.pallas_call(
        _topk_body,
        grid=(1,),
        in_specs=[pl.BlockSpec((_N, _B), lambda i: (0, 0))],
        out_specs=[
            pl.BlockSpec((_K, _B), lambda i: (0, 0)),
            pl.BlockSpec((_K, _B), lambda i: (0, 0)),
        ],
        out_shape=[
            jax.ShapeDtypeStruct((_K, _B), jnp.float32),
            jax.ShapeDtypeStruct((_K, _B), jnp.int32),
        ],
    )(score_t)


def kernel(grounding_x, rule_weights, bias, all_h, all_r, k):
    gx_t = jnp.transpose(grounding_x, (0, 2, 1))
    wb = rule_weights.astype(jnp.bfloat16).reshape(1, _R)
    bias2 = bias.reshape(_N, 1)
    score_t = _score(gx_t, wb, bias2)
    vals_t, idx_t = _topk(score_t)
    mask = jnp.ones((_B, _N), dtype=jnp.bool_)
    return score_t.T, mask, vals_t.T, idx_t.T


# speedup vs baseline: 49.3340x; 1.0428x over previous
import functools

import jax
import jax.numpy as jnp
from jax import lax
from jax.experimental import pallas as pl
from jax.experimental.pallas import tpu as pltpu

_R = 64
_B = 128
_N = 10000
_K = 20

_NB = 400
_NBLK = _N // _NB


def _score_body(w_ref, gx_ref, bias_ref, out_ref):
    xb = gx_ref[...].astype(jnp.bfloat16)
    x2 = xb.reshape(_R, _NB * _B)
    wb = w_ref[...]
    acc = lax.dot_general(
        wb, x2, (((1,), (0,)), ((), ())),
        preferred_element_type=jnp.float32,
    ).reshape(_NB, _B)
    out_ref[...] = acc + bias_ref[...]


def _score(gx_t, wb, bias2):
    return pl.pallas_call(
        _score_body,
        grid=(_NBLK,),
        in_specs=[
            pl.BlockSpec((1, _R), lambda j: (0, 0)),
            pl.BlockSpec((_R, _NB, _B), lambda j: (0, j, 0)),
            pl.BlockSpec((_NB, 1), lambda j: (j, 0)),
        ],
        out_specs=pl.BlockSpec((_NB, _B), lambda j: (j, 0)),
        out_shape=jax.ShapeDtypeStruct((_N, _B), jnp.float32),
        compiler_params=pltpu.CompilerParams(
            dimension_semantics=("arbitrary",),
        ),
    )(wb, gx_t, bias2)


def _topk_body(s_ref, vals_ref, idx_ref):
    s = s_ref[...]
    row = lax.broadcasted_iota(jnp.int32, (_N, _B), 0)
    krow = lax.broadcasted_iota(jnp.int32, (_K, _B), 0)
    vals = jnp.zeros((_K, _B), jnp.float32)
    idxs = jnp.zeros((_K, _B), jnp.int32)
    for i in range(_K):
        m = jnp.max(s, axis=0, keepdims=True)
        cand = jnp.where(s == m, row, jnp.int32(2**30))
        ix = jnp.min(cand, axis=0, keepdims=True)
        vals = jnp.where(krow == i, m, vals)
        idxs = jnp.where(krow == i, ix, idxs)
        s = jnp.where(row == ix, -jnp.inf, s)
    vals_ref[...] = vals
    idx_ref[...] = idxs


def _topk(score_t):
    return pl.pallas_call(
        _topk_body,
        grid=(1,),
        in_specs=[pl.BlockSpec((_N, _B), lambda i: (0, 0))],
        out_specs=[
            pl.BlockSpec((_K, _B), lambda i: (0, 0)),
            pl.BlockSpec((_K, _B), lambda i: (0, 0)),
        ],
        out_shape=[
            jax.ShapeDtypeStruct((_K, _B), jnp.float32),
            jax.ShapeDtypeStruct((_K, _B), jnp.int32),
        ],
    )(score_t)


def kernel(grounding_x, rule_weights, bias, all_h, all_r, k):
    gx_t = jnp.transpose(grounding_x, (0, 2, 1))
    wb = rule_weights.astype(jnp.bfloat16).reshape(1, _R)
    bias2 = bias.reshape(_N, 1)
    score_t = _score(gx_t, wb, bias2)
    vals_t, idx_t = _topk(score_t)
    mask = jnp.ones((_B, _N), dtype=jnp.bool_)
    return score_t.T, mask, vals_t.T, idx_t.T
